# Initial kernel scaffold; baseline (speedup 1.0000x reference)
import jax
import jax.numpy as jnp
from jax.experimental import pallas as pl


def kernel(loc_preds, conf_preds, priors, gt_boxes, gt_labels):
    raise NotImplementedError("write your pallas kernel here")


# speedup vs baseline: 12.3513x; 12.3513x over previous
import jax
import jax.numpy as jnp
from jax.experimental import pallas as pl
from jax.experimental.pallas import tpu as pltpu

_IOU_THRESHOLD = 0.5
_NEG_POS_RATIO = 3.0
_BISECT_ITERS = 40


def _row_body(loc_ref, conf_ref, pri_ref, box_ref, lbl_ref, out_ref):
    f32 = jnp.float32
    num_p = loc_ref.shape[2]
    num_g = box_ref.shape[1]
    num_c = conf_ref.shape[1]

    pcx = pri_ref[0:1, :]
    pcy = pri_ref[1:2, :]
    pw = pri_ref[2:3, :]
    ph = pri_ref[3:4, :]
    px1 = pcx - pw * 0.5
    py1 = pcy - ph * 0.5
    px2 = pcx + pw * 0.5
    py2 = pcy + ph * 0.5

    boxes = box_ref[0]
    gx1 = boxes[:, 0:1]
    gy1 = boxes[:, 1:2]
    gx2 = boxes[:, 2:3]
    gy2 = boxes[:, 3:4]
    lblf = lbl_ref[0]

    ltx = jnp.maximum(px1, gx1)
    lty = jnp.maximum(py1, gy1)
    rbx = jnp.minimum(px2, gx2)
    rby = jnp.minimum(py2, gy2)
    iw = jnp.maximum(rbx - ltx, 0.0)
    ih = jnp.maximum(rby - lty, 0.0)
    inter = iw * ih
    area_p = (px2 - px1) * (py2 - py1)
    area_g = (gx2 - gx1) * (gy2 - gy1)
    ov = inter / (area_p + area_g - inter)

    g_iota = jax.lax.broadcasted_iota(jnp.int32, (num_g, num_p), 0).astype(f32)
    p_iota = jax.lax.broadcasted_iota(jnp.int32, (num_g, num_p), 1).astype(f32)

    bov = jnp.max(ov, axis=0, keepdims=True)
    bgi = jnp.min(jnp.where(ov == bov, g_iota, float(num_g)), axis=0,
                  keepdims=True)
    m_g = jnp.max(ov, axis=1, keepdims=True)
    bpi = jnp.min(jnp.where(ov == m_g, p_iota, float(num_p)), axis=1,
                  keepdims=True)

    claimed_g = jnp.max(jnp.where(p_iota == bpi, g_iota, -1.0), axis=0,
                        keepdims=True)
    claimed = claimed_g >= 0.0
    bgi = jnp.where(claimed, claimed_g, bgi)
    bov = jnp.where(claimed, 1.0, bov)

    pos = bov > _IOU_THRESHOLD
    posf = pos.astype(f32)
    pos_cnt = jnp.sum(posf)

    onehot = g_iota == bgi
    gcx = (gx1 + gx2) * 0.5
    gcy = (gy1 + gy2) * 0.5
    gw = gx2 - gx1
    gh = gy2 - gy1
    mcx = jnp.sum(jnp.where(onehot, gcx, 0.0), axis=0, keepdims=True)
    mcy = jnp.sum(jnp.where(onehot, gcy, 0.0), axis=0, keepdims=True)
    mw = jnp.sum(jnp.where(onehot, gw, 0.0), axis=0, keepdims=True)
    mh = jnp.sum(jnp.where(onehot, gh, 0.0), axis=0, keepdims=True)
    tgt_cls = jnp.where(pos,
                        jnp.sum(jnp.where(onehot, lblf, 0.0), axis=0,
                                keepdims=True),
                        0.0)

    e0 = (mcx - pcx) / pw
    e1 = (mcy - pcy) / ph
    e2 = jnp.log(mw / pw)
    e3 = jnp.log(mh / ph)
    loc_sum = 0.0
    for c, enc in enumerate((e0, e1, e2, e3)):
        d = loc_ref[0, c:c + 1, :] - enc
        ad = jnp.abs(d)
        sl1 = jnp.where(ad < 1.0, 0.5 * d * d, ad - 0.5)
        loc_sum = loc_sum + jnp.sum(jnp.where(pos, sl1, 0.0))

    x = conf_ref[0]
    xm = jnp.max(x, axis=0, keepdims=True)
    lse = jnp.log(jnp.sum(jnp.exp(x - xm), axis=0, keepdims=True)) + xm
    c_iota = jax.lax.broadcasted_iota(jnp.int32, (num_c, num_p), 0).astype(f32)
    tgt_logit = jnp.sum(jnp.where(c_iota == tgt_cls, x, 0.0), axis=0,
                        keepdims=True)
    ce = lse - tgt_logit

    conf_pos_sum = jnp.sum(jnp.where(pos, ce, 0.0))
    ce_neg = jnp.where(pos, 0.0, ce)

    k = jnp.maximum(pos_cnt * _NEG_POS_RATIO, 1.0)

    def _bis(_, carry):
        lo, hi = carry
        t = 0.5 * (lo + hi)
        n = jnp.sum((ce_neg > t).astype(f32))
        gt = n > k
        return jnp.where(gt, t, lo), jnp.where(gt, hi, t)

    lo0 = jnp.float32(0.0)
    hi0 = jnp.max(ce_neg)
    _, thr = jax.lax.fori_loop(0, _BISECT_ITERS, _bis, (lo0, hi0))
    above = ce_neg > thr
    n_above = jnp.sum(above.astype(f32))
    s_above = jnp.sum(jnp.where(above, ce_neg, 0.0))
    neg_sum = s_above + (k - n_above) * thr

    o_iota = jax.lax.broadcasted_iota(jnp.int32, (1, 8), 1)
    out_ref[0] = jnp.where(
        o_iota == 0, loc_sum,
        jnp.where(o_iota == 1, pos_cnt,
                  jnp.where(o_iota == 2, conf_pos_sum,
                            jnp.where(o_iota == 3, neg_sum, 0.0))))


def kernel(loc_preds, conf_preds, priors, gt_boxes, gt_labels):
    b, p, _ = loc_preds.shape
    c = conf_preds.shape[2]
    g = gt_boxes.shape[1]
    loc_t = jnp.transpose(loc_preds, (0, 2, 1))
    conf_t = jnp.transpose(conf_preds, (0, 2, 1))
    pri_t = jnp.transpose(priors, (1, 0))
    lbl = gt_labels.astype(jnp.float32).reshape(b, g, 1)

    sums = pl.pallas_call(
        _row_body,
        grid=(b,),
        in_specs=[
            pl.BlockSpec((1, 4, p), lambda i: (i, 0, 0)),
            pl.BlockSpec((1, c, p), lambda i: (i, 0, 0)),
            pl.BlockSpec((4, p), lambda i: (0, 0)),
            pl.BlockSpec((1, g, 4), lambda i: (i, 0, 0)),
            pl.BlockSpec((1, g, 1), lambda i: (i, 0, 0)),
        ],
        out_specs=pl.BlockSpec((1, 1, 8), lambda i: (i, 0, 0)),
        out_shape=jax.ShapeDtypeStruct((b, 1, 8), jnp.float32),
        compiler_params=pltpu.CompilerParams(
            dimension_semantics=("parallel",)),
    )(loc_t, conf_t, pri_t, gt_boxes, lbl)

    loc_sum = jnp.sum(sums[:, 0, 0])
    num_pos = jnp.maximum(jnp.sum(sums[:, 0, 1]), 1.0)
    conf_sum = jnp.sum(sums[:, 0, 2]) + jnp.sum(sums[:, 0, 3])
    return (loc_sum + conf_sum) / num_pos


# speedup vs baseline: 18.8181x; 1.5236x over previous
import jax
import jax.numpy as jnp
from jax.experimental import pallas as pl
from jax.experimental.pallas import tpu as pltpu

_IOU_THRESHOLD = 0.5
_NEG_POS_RATIO = 3.0
_BISECT_ITERS = 22


def _row_body(loc_ref, conf_ref, pri_ref, box_ref, attr_ref, out_ref):
    f32 = jnp.float32
    num_p = loc_ref.shape[2]
    num_g = box_ref.shape[1]
    num_c = conf_ref.shape[1]

    pcx = pri_ref[0:1, :]
    pcy = pri_ref[1:2, :]
    pw = pri_ref[2:3, :]
    ph = pri_ref[3:4, :]
    px1 = pcx - pw * 0.5
    py1 = pcy - ph * 0.5
    px2 = pcx + pw * 0.5
    py2 = pcy + ph * 0.5

    boxes = box_ref[0]
    gx1 = boxes[:, 0:1]
    gy1 = boxes[:, 1:2]
    gx2 = boxes[:, 2:3]
    gy2 = boxes[:, 3:4]

    ltx = jnp.maximum(px1, gx1)
    lty = jnp.maximum(py1, gy1)
    rbx = jnp.minimum(px2, gx2)
    rby = jnp.minimum(py2, gy2)
    iw = jnp.maximum(rbx - ltx, 0.0)
    ih = jnp.maximum(rby - lty, 0.0)
    inter = iw * ih
    area_p = (px2 - px1) * (py2 - py1)
    area_g = (gx2 - gx1) * (gy2 - gy1)
    ov = inter / (area_p + area_g - inter)

    g_iota = jax.lax.broadcasted_iota(jnp.int32, (num_g, num_p), 0).astype(f32)
    p_iota = jax.lax.broadcasted_iota(jnp.int32, (num_g, num_p), 1).astype(f32)

    bov = jnp.max(ov, axis=0, keepdims=True)
    bgi = jnp.min(jnp.where(ov == bov, g_iota, float(num_g)), axis=0,
                  keepdims=True)
    m_g = jnp.max(ov, axis=1, keepdims=True)
    bpi = jnp.min(jnp.where(ov == m_g, p_iota, float(num_p)), axis=1,
                  keepdims=True)

    claimed_g = jnp.max(jnp.where(p_iota == bpi, g_iota, -1.0), axis=0,
                        keepdims=True)
    claimed = claimed_g >= 0.0
    bgi = jnp.where(claimed, claimed_g, bgi)
    bov = jnp.where(claimed, 1.0, bov)

    pos = bov > _IOU_THRESHOLD
    posf = pos.astype(f32)
    pos_cnt = jnp.sum(posf)

    onehot_f = jnp.where(g_iota == bgi, 1.0, 0.0)
    m = jax.lax.dot_general(attr_ref[0], onehot_f,
                            dimension_numbers=(((1,), (0,)), ((), ())),
                            preferred_element_type=jnp.float32)
    mcx = (m[0:1, :] + m[2:3, :]) * 0.5
    mcy = (m[1:2, :] + m[3:4, :]) * 0.5
    mw = m[2:3, :] - m[0:1, :]
    mh = m[3:4, :] - m[1:2, :]
    tgt_cls = jnp.where(pos, m[4:5, :], 0.0)

    e0 = (mcx - pcx) / pw
    e1 = (mcy - pcy) / ph
    e2 = jnp.log(mw / pw)
    e3 = jnp.log(mh / ph)
    loc_sum = 0.0
    for c, enc in enumerate((e0, e1, e2, e3)):
        d = loc_ref[0, c:c + 1, :] - enc
        ad = jnp.abs(d)
        sl1 = jnp.where(ad < 1.0, 0.5 * d * d, ad - 0.5)
        loc_sum = loc_sum + jnp.sum(jnp.where(pos, sl1, 0.0))

    x = conf_ref[0]
    xm = jnp.max(x, axis=0, keepdims=True)
    lse = jnp.log(jnp.sum(jnp.exp(x - xm), axis=0, keepdims=True)) + xm
    c_iota = jax.lax.broadcasted_iota(jnp.int32, (num_c, num_p), 0).astype(f32)
    tgt_logit = jnp.sum(jnp.where(c_iota == tgt_cls, x, 0.0), axis=0,
                        keepdims=True)
    ce = lse - tgt_logit

    conf_pos_sum = jnp.sum(jnp.where(pos, ce, 0.0))
    ce_neg = jnp.where(pos, 0.0, ce)

    k = jnp.maximum(pos_cnt * _NEG_POS_RATIO, 1.0)

    def _bis(_, carry):
        lo, hi = carry
        t = 0.5 * (lo + hi)
        n = jnp.sum((ce_neg > t).astype(f32))
        gt = n > k
        return jnp.where(gt, t, lo), jnp.where(gt, hi, t)

    lo0 = jnp.float32(0.0)
    hi0 = jnp.max(ce_neg)
    _, thr = jax.lax.fori_loop(0, _BISECT_ITERS, _bis, (lo0, hi0))
    above = ce_neg > thr
    n_above = jnp.sum(above.astype(f32))
    s_above = jnp.sum(jnp.where(above, ce_neg, 0.0))
    neg_sum = s_above + (k - n_above) * thr

    o_iota = jax.lax.broadcasted_iota(jnp.int32, (1, 8), 1)
    out_ref[0] = jnp.where(
        o_iota == 0, loc_sum,
        jnp.where(o_iota == 1, pos_cnt,
                  jnp.where(o_iota == 2, conf_pos_sum,
                            jnp.where(o_iota == 3, neg_sum, 0.0))))


def kernel(loc_preds, conf_preds, priors, gt_boxes, gt_labels):
    b, p, _ = loc_preds.shape
    c = conf_preds.shape[2]
    g = gt_boxes.shape[1]
    loc_t = jnp.transpose(loc_preds, (0, 2, 1))
    conf_t = jnp.transpose(conf_preds, (0, 2, 1))
    pri_t = jnp.transpose(priors, (1, 0))
    attrs = jnp.concatenate(
        [jnp.transpose(gt_boxes, (0, 2, 1)),
         gt_labels.astype(jnp.float32)[:, None, :]], axis=1)

    sums = pl.pallas_call(
        _row_body,
        grid=(b,),
        in_specs=[
            pl.BlockSpec((1, 4, p), lambda i: (i, 0, 0)),
            pl.BlockSpec((1, c, p), lambda i: (i, 0, 0)),
            pl.BlockSpec((4, p), lambda i: (0, 0)),
            pl.BlockSpec((1, g, 4), lambda i: (i, 0, 0)),
            pl.BlockSpec((1, 5, g), lambda i: (i, 0, 0)),
        ],
        out_specs=pl.BlockSpec((1, 1, 8), lambda i: (i, 0, 0)),
        out_shape=jax.ShapeDtypeStruct((b, 1, 8), jnp.float32),
        compiler_params=pltpu.CompilerParams(
            dimension_semantics=("parallel",)),
    )(loc_t, conf_t, pri_t, gt_boxes, attrs)

    loc_sum = jnp.sum(sums[:, 0, 0])
    num_pos = jnp.maximum(jnp.sum(sums[:, 0, 1]), 1.0)
    conf_sum = jnp.sum(sums[:, 0, 2]) + jnp.sum(sums[:, 0, 3])
    return (loc_sum + conf_sum) / num_pos


# speedup vs baseline: 22.7378x; 1.2083x over previous
import functools

import jax
import jax.numpy as jnp
from jax.experimental import pallas as pl
from jax.experimental.pallas import tpu as pltpu

_IOU_THRESHOLD = 0.5
_NEG_POS_RATIO = 3.0
_BISECT_ITERS = 20
_LANES = 128
_SUBLANES = 72


def _row_body(num_p_real, loc_ref, conf_ref, pri_ref, box_ref, lbl_ref,
              out_ref):
    f32 = jnp.float32
    ss = pri_ref.shape[1]
    ll = pri_ref.shape[2]
    num_g = box_ref.shape[1]
    num_c = conf_ref.shape[1]

    pcx = pri_ref[0:1]
    pcy = pri_ref[1:2]
    pw = pri_ref[2:3]
    ph = pri_ref[3:4]
    px1 = pcx - pw * 0.5
    py1 = pcy - ph * 0.5
    px2 = pcx + pw * 0.5
    py2 = pcy + ph * 0.5

    boxes = box_ref[0]
    gx1 = boxes[:, 0:1][..., None]
    gy1 = boxes[:, 1:2][..., None]
    gx2 = boxes[:, 2:3][..., None]
    gy2 = boxes[:, 3:4][..., None]
    lblf = lbl_ref[0][..., None]

    ltx = jnp.maximum(px1, gx1)
    lty = jnp.maximum(py1, gy1)
    rbx = jnp.minimum(px2, gx2)
    rby = jnp.minimum(py2, gy2)
    iw = jnp.maximum(rbx - ltx, 0.0)
    ih = jnp.maximum(rby - lty, 0.0)
    inter = iw * ih
    area_p = (px2 - px1) * (py2 - py1)
    area_g = (gx2 - gx1) * (gy2 - gy1)
    ov = inter / (area_p + area_g - inter)

    s_io = jax.lax.broadcasted_iota(jnp.int32, (1, ss, ll), 1)
    l_io = jax.lax.broadcasted_iota(jnp.int32, (1, ss, ll), 2)
    p_iota = (s_io * ll + l_io).astype(f32)
    valid = p_iota < float(num_p_real)
    g_iota = jax.lax.broadcasted_iota(jnp.int32, (num_g, 1, 1), 0).astype(f32)

    bov = jnp.max(ov, axis=0, keepdims=True)
    bgi = jnp.min(jnp.where(ov == bov, g_iota, float(num_g)), axis=0,
                  keepdims=True)
    m_g = jnp.max(ov, axis=(1, 2), keepdims=True)
    bpi = jnp.min(jnp.where(ov == m_g, p_iota, float(ss * ll)), axis=(1, 2),
                  keepdims=True)

    claimed_g = jnp.max(jnp.where(p_iota == bpi, g_iota, -1.0), axis=0,
                        keepdims=True)
    claimed = claimed_g >= 0.0
    bgi = jnp.where(claimed, claimed_g, bgi)
    bov = jnp.where(claimed, 1.0, bov)

    pos = bov > _IOU_THRESHOLD
    pos_cnt = jnp.sum(pos.astype(f32))

    onehot = g_iota == bgi
    gcx = (gx1 + gx2) * 0.5
    gcy = (gy1 + gy2) * 0.5
    gw = gx2 - gx1
    gh = gy2 - gy1
    mcx = jnp.sum(jnp.where(onehot, gcx, 0.0), axis=0, keepdims=True)
    mcy = jnp.sum(jnp.where(onehot, gcy, 0.0), axis=0, keepdims=True)
    mw = jnp.sum(jnp.where(onehot, gw, 0.0), axis=0, keepdims=True)
    mh = jnp.sum(jnp.where(onehot, gh, 0.0), axis=0, keepdims=True)
    tgt_cls = jnp.where(pos,
                        jnp.sum(jnp.where(onehot, lblf, 0.0), axis=0,
                                keepdims=True),
                        0.0)

    e0 = (mcx - pcx) / pw
    e1 = (mcy - pcy) / ph
    e2 = jnp.log(mw / pw)
    e3 = jnp.log(mh / ph)
    sl1_acc = jnp.zeros((1, ss, ll), f32)
    for c, enc in enumerate((e0, e1, e2, e3)):
        d = loc_ref[0, c:c + 1] - enc
        ad = jnp.abs(d)
        sl1 = jnp.where(ad < 1.0, 0.5 * d * d, ad - 0.5)
        sl1_acc = sl1_acc + jnp.where(pos, sl1, 0.0)
    loc_sum = jnp.sum(sl1_acc)

    x = conf_ref[0]
    xm = jnp.max(x, axis=0, keepdims=True)
    lse = jnp.log(jnp.sum(jnp.exp(x - xm), axis=0, keepdims=True)) + xm
    c_iota = jax.lax.broadcasted_iota(jnp.int32, (num_c, 1, 1), 0).astype(f32)
    tgt_logit = jnp.sum(jnp.where(c_iota == tgt_cls, x, 0.0), axis=0,
                        keepdims=True)
    ce = lse - tgt_logit

    conf_pos_sum = jnp.sum(jnp.where(pos, ce, 0.0))
    ce_neg = jnp.where(jnp.logical_or(pos, jnp.logical_not(valid)), 0.0, ce)

    k = jnp.maximum(pos_cnt * _NEG_POS_RATIO, 1.0)

    def _bis(_, carry):
        lo, hi = carry
        t = 0.5 * (lo + hi)
        n = jnp.sum((ce_neg > t).astype(f32))
        gt = n > k
        return jnp.where(gt, t, lo), jnp.where(gt, hi, t)

    lo0 = jnp.float32(0.0)
    hi0 = jnp.max(ce_neg)
    _, thr = jax.lax.fori_loop(0, _BISECT_ITERS, _bis, (lo0, hi0))
    above = ce_neg > thr
    n_above = jnp.sum(above.astype(f32))
    s_above = jnp.sum(jnp.where(above, ce_neg, 0.0))
    neg_sum = s_above + (k - n_above) * thr

    o_iota = jax.lax.broadcasted_iota(jnp.int32, (1, 8), 1)
    out_ref[0] = jnp.where(
        o_iota == 0, loc_sum,
        jnp.where(o_iota == 1, pos_cnt,
                  jnp.where(o_iota == 2, conf_pos_sum,
                            jnp.where(o_iota == 3, neg_sum, 0.0))))


def kernel(loc_preds, conf_preds, priors, gt_boxes, gt_labels):
    b, p, _ = loc_preds.shape
    c = conf_preds.shape[2]
    g = gt_boxes.shape[1]
    pp = _SUBLANES * _LANES
    pad = pp - p

    loc_t = jnp.pad(jnp.transpose(loc_preds, (0, 2, 1)),
                    ((0, 0), (0, 0), (0, pad))).reshape(b, 4, _SUBLANES,
                                                        _LANES)
    conf_t = jnp.pad(jnp.transpose(conf_preds, (0, 2, 1)),
                     ((0, 0), (0, 0), (0, pad))).reshape(b, c, _SUBLANES,
                                                         _LANES)
    pri_t = jnp.pad(jnp.transpose(priors, (1, 0)),
                    ((0, 0), (0, pad))).reshape(4, _SUBLANES, _LANES)
    lbl = gt_labels.astype(jnp.float32).reshape(b, g, 1)

    sums = pl.pallas_call(
        functools.partial(_row_body, p),
        grid=(b,),
        in_specs=[
            pl.BlockSpec((1, 4, _SUBLANES, _LANES), lambda i: (i, 0, 0, 0)),
            pl.BlockSpec((1, c, _SUBLANES, _LANES), lambda i: (i, 0, 0, 0)),
            pl.BlockSpec((4, _SUBLANES, _LANES), lambda i: (0, 0, 0)),
            pl.BlockSpec((1, g, 4), lambda i: (i, 0, 0)),
            pl.BlockSpec((1, g, 1), lambda i: (i, 0, 0)),
        ],
        out_specs=pl.BlockSpec((1, 1, 8), lambda i: (i, 0, 0)),
        out_shape=jax.ShapeDtypeStruct((b, 1, 8), jnp.float32),
        compiler_params=pltpu.CompilerParams(
            dimension_semantics=("parallel",)),
    )(loc_t, conf_t, pri_t, gt_boxes, lbl)

    loc_sum = jnp.sum(sums[:, 0, 0])
    num_pos = jnp.maximum(jnp.sum(sums[:, 0, 1]), 1.0)
    conf_sum = jnp.sum(sums[:, 0, 2]) + jnp.sum(sums[:, 0, 3])
    return (loc_sum + conf_sum) / num_pos


# speedup vs baseline: 24.9709x; 1.0982x over previous
import functools

import jax
import jax.numpy as jnp
from jax.experimental import pallas as pl
from jax.experimental.pallas import tpu as pltpu

_IOU_THRESHOLD = 0.5
_NEG_POS_RATIO = 3.0
_BISECT_ITERS = 16
_LANES = 128
_SUBLANES = 72
_IMGS_PER_PROG = 2


def _one_image(num_p_real, loc, conf, pri, boxes, lblf):
    f32 = jnp.float32
    _, ss, ll = pri.shape
    num_g = boxes.shape[0]
    num_c = conf.shape[0]

    pcx = pri[0:1]
    pcy = pri[1:2]
    pw = pri[2:3]
    ph = pri[3:4]
    px1 = pcx - pw * 0.5
    py1 = pcy - ph * 0.5
    px2 = pcx + pw * 0.5
    py2 = pcy + ph * 0.5

    gx1 = boxes[:, 0:1][..., None]
    gy1 = boxes[:, 1:2][..., None]
    gx2 = boxes[:, 2:3][..., None]
    gy2 = boxes[:, 3:4][..., None]

    ltx = jnp.maximum(px1, gx1)
    lty = jnp.maximum(py1, gy1)
    rbx = jnp.minimum(px2, gx2)
    rby = jnp.minimum(py2, gy2)
    iw = jnp.maximum(rbx - ltx, 0.0)
    ih = jnp.maximum(rby - lty, 0.0)
    inter = iw * ih
    area_p = (px2 - px1) * (py2 - py1)
    area_g = (gx2 - gx1) * (gy2 - gy1)
    ov = inter / (area_p + area_g - inter)

    s_io = jax.lax.broadcasted_iota(jnp.int32, (1, ss, ll), 1)
    l_io = jax.lax.broadcasted_iota(jnp.int32, (1, ss, ll), 2)
    p_iota = (s_io * ll + l_io).astype(f32)
    valid = p_iota < float(num_p_real)
    g_iota = jax.lax.broadcasted_iota(jnp.int32, (num_g, 1, 1), 0).astype(f32)

    bov = jnp.max(ov, axis=0, keepdims=True)
    bgi = jnp.min(jnp.where(ov == bov, g_iota, float(num_g)), axis=0,
                  keepdims=True)
    m_g = jnp.max(ov, axis=(1, 2), keepdims=True)
    bpi = jnp.min(jnp.where(ov == m_g, p_iota, float(ss * ll)), axis=(1, 2),
                  keepdims=True)

    claimed_g = jnp.max(jnp.where(p_iota == bpi, g_iota, -1.0), axis=0,
                        keepdims=True)
    claimed = claimed_g >= 0.0
    bgi = jnp.where(claimed, claimed_g, bgi)
    bov = jnp.where(claimed, 1.0, bov)

    pos = bov > _IOU_THRESHOLD
    pos_cnt = jnp.sum(pos.astype(f32))

    onehot = g_iota == bgi
    gcx = (gx1 + gx2) * 0.5
    gcy = (gy1 + gy2) * 0.5
    gw = gx2 - gx1
    gh = gy2 - gy1
    mcx = jnp.sum(jnp.where(onehot, gcx, 0.0), axis=0, keepdims=True)
    mcy = jnp.sum(jnp.where(onehot, gcy, 0.0), axis=0, keepdims=True)
    mw = jnp.sum(jnp.where(onehot, gw, 0.0), axis=0, keepdims=True)
    mh = jnp.sum(jnp.where(onehot, gh, 0.0), axis=0, keepdims=True)
    tgt_cls = jnp.where(pos,
                        jnp.sum(jnp.where(onehot, lblf, 0.0), axis=0,
                                keepdims=True),
                        0.0)

    e0 = (mcx - pcx) / pw
    e1 = (mcy - pcy) / ph
    e2 = jnp.log(mw / pw)
    e3 = jnp.log(mh / ph)
    sl1_acc = jnp.zeros((1, ss, ll), f32)
    for c, enc in enumerate((e0, e1, e2, e3)):
        d = loc[c:c + 1] - enc
        ad = jnp.abs(d)
        sl1 = jnp.where(ad < 1.0, 0.5 * d * d, ad - 0.5)
        sl1_acc = sl1_acc + jnp.where(pos, sl1, 0.0)
    loc_sum = jnp.sum(sl1_acc)

    xm = jnp.max(conf, axis=0, keepdims=True)
    lse = jnp.log(jnp.sum(jnp.exp(conf - xm), axis=0, keepdims=True)) + xm
    c_iota = jax.lax.broadcasted_iota(jnp.int32, (num_c, 1, 1), 0).astype(f32)
    tgt_logit = jnp.sum(jnp.where(c_iota == tgt_cls, conf, 0.0), axis=0,
                        keepdims=True)
    ce = lse - tgt_logit

    conf_pos_sum = jnp.sum(jnp.where(pos, ce, 0.0))
    ce_neg = jnp.where(jnp.logical_or(pos, jnp.logical_not(valid)), 0.0, ce)

    k = jnp.maximum(pos_cnt * _NEG_POS_RATIO, 1.0)

    def _bis(_, carry):
        lo, hi = carry
        t = 0.5 * (lo + hi)
        n = jnp.sum((ce_neg > t).astype(f32))
        gt = n > k
        return jnp.where(gt, t, lo), jnp.where(gt, hi, t)

    lo0 = jnp.float32(0.0)
    hi0 = jnp.max(ce_neg)
    _, thr = jax.lax.fori_loop(0, _BISECT_ITERS, _bis, (lo0, hi0))
    above = ce_neg > thr
    n_above = jnp.sum(above.astype(f32))
    s_above = jnp.sum(jnp.where(above, ce_neg, 0.0))
    neg_sum = s_above + (k - n_above) * thr

    return loc_sum, pos_cnt, conf_pos_sum, neg_sum


def _prog_body(num_p_real, loc_ref, conf_ref, pri_ref, box_ref, lbl_ref,
               out_ref):
    pri = pri_ref[...]
    o_iota = jax.lax.broadcasted_iota(jnp.int32, (1, 8), 1)
    for j in range(loc_ref.shape[0]):
        loc_sum, pos_cnt, conf_pos_sum, neg_sum = _one_image(
            num_p_real, loc_ref[j], conf_ref[j], pri, box_ref[j],
            lbl_ref[j][..., None])
        out_ref[j] = jnp.where(
            o_iota == 0, loc_sum,
            jnp.where(o_iota == 1, pos_cnt,
                      jnp.where(o_iota == 2, conf_pos_sum,
                                jnp.where(o_iota == 3, neg_sum, 0.0))))


def kernel(loc_preds, conf_preds, priors, gt_boxes, gt_labels):
    b, p, _ = loc_preds.shape
    c = conf_preds.shape[2]
    g = gt_boxes.shape[1]
    pp = _SUBLANES * _LANES
    pad = pp - p
    ipp = _IMGS_PER_PROG

    loc_t = jnp.pad(jnp.transpose(loc_preds, (0, 2, 1)),
                    ((0, 0), (0, 0), (0, pad))).reshape(b, 4, _SUBLANES,
                                                        _LANES)
    conf_t = jnp.pad(jnp.transpose(conf_preds, (0, 2, 1)),
                     ((0, 0), (0, 0), (0, pad))).reshape(b, c, _SUBLANES,
                                                         _LANES)
    pri_t = jnp.pad(jnp.transpose(priors, (1, 0)),
                    ((0, 0), (0, pad))).reshape(4, _SUBLANES, _LANES)
    lbl = gt_labels.astype(jnp.float32).reshape(b, g, 1)

    sums = pl.pallas_call(
        functools.partial(_prog_body, p),
        grid=(b // ipp,),
        in_specs=[
            pl.BlockSpec((ipp, 4, _SUBLANES, _LANES),
                         lambda i: (i, 0, 0, 0)),
            pl.BlockSpec((ipp, c, _SUBLANES, _LANES),
                         lambda i: (i, 0, 0, 0)),
            pl.BlockSpec((4, _SUBLANES, _LANES), lambda i: (0, 0, 0)),
            pl.BlockSpec((ipp, g, 4), lambda i: (i, 0, 0)),
            pl.BlockSpec((ipp, g, 1), lambda i: (i, 0, 0)),
        ],
        out_specs=pl.BlockSpec((ipp, 1, 8), lambda i: (i, 0, 0)),
        out_shape=jax.ShapeDtypeStruct((b, 1, 8), jnp.float32),
        compiler_params=pltpu.CompilerParams(
            dimension_semantics=("parallel",)),
    )(loc_t, conf_t, pri_t, gt_boxes, lbl)

    loc_sum = jnp.sum(sums[:, 0, 0])
    num_pos = jnp.maximum(jnp.sum(sums[:, 0, 1]), 1.0)
    conf_sum = jnp.sum(sums[:, 0, 2]) + jnp.sum(sums[:, 0, 3])
    return (loc_sum + conf_sum) / num_pos


# speedup vs baseline: 36.3169x; 1.4544x over previous
import functools

import jax
import jax.numpy as jnp
from jax.experimental import pallas as pl
from jax.experimental.pallas import tpu as pltpu

_IOU_THRESHOLD = 0.5
_NEG_POS_RATIO = 3.0
_BISECT_ITERS = 16
_LANES = 128
_SUBLANES = 72
_IMGS_PER_PROG = 4


def _one_image(num_p_real, loc, conf, pri, boxes, lblf):
    f32 = jnp.float32
    _, ss, ll = pri.shape
    num_g = boxes.shape[0]
    num_c = conf.shape[0]

    pcx = pri[0:1]
    pcy = pri[1:2]
    pw = pri[2:3]
    ph = pri[3:4]
    px1 = pcx - pw * 0.5
    py1 = pcy - ph * 0.5
    px2 = pcx + pw * 0.5
    py2 = pcy + ph * 0.5

    gx1 = boxes[:, 0:1][..., None]
    gy1 = boxes[:, 1:2][..., None]
    gx2 = boxes[:, 2:3][..., None]
    gy2 = boxes[:, 3:4][..., None]

    ltx = jnp.maximum(px1, gx1)
    lty = jnp.maximum(py1, gy1)
    rbx = jnp.minimum(px2, gx2)
    rby = jnp.minimum(py2, gy2)
    iw = jnp.maximum(rbx - ltx, 0.0)
    ih = jnp.maximum(rby - lty, 0.0)
    inter = iw * ih
    area_p = (px2 - px1) * (py2 - py1)
    area_g = (gx2 - gx1) * (gy2 - gy1)
    ov = inter / (area_p + area_g - inter)

    s_io = jax.lax.broadcasted_iota(jnp.int32, (1, ss, ll), 1)
    l_io = jax.lax.broadcasted_iota(jnp.int32, (1, ss, ll), 2)
    p_iota = (s_io * ll + l_io).astype(f32)
    valid = p_iota < float(num_p_real)
    g_iota = jax.lax.broadcasted_iota(jnp.int32, (num_g, 1, 1), 0).astype(f32)

    bov = jnp.max(ov, axis=0, keepdims=True)
    bgi = jnp.min(jnp.where(ov == bov, g_iota, float(num_g)), axis=0,
                  keepdims=True)
    m_g = jnp.max(ov, axis=(1, 2), keepdims=True)
    bpi = jnp.min(jnp.where(ov == m_g, p_iota, float(ss * ll)), axis=(1, 2),
                  keepdims=True)

    claimed_g = jnp.max(jnp.where(p_iota == bpi, g_iota, -1.0), axis=0,
                        keepdims=True)
    claimed = claimed_g >= 0.0
    bgi = jnp.where(claimed, claimed_g, bgi)
    bov = jnp.where(claimed, 1.0, bov)

    pos = bov > _IOU_THRESHOLD

    onehot = g_iota == bgi
    gcx = (gx1 + gx2) * 0.5
    gcy = (gy1 + gy2) * 0.5
    gw = gx2 - gx1
    gh = gy2 - gy1
    mcx = jnp.sum(jnp.where(onehot, gcx, 0.0), axis=0, keepdims=True)
    mcy = jnp.sum(jnp.where(onehot, gcy, 0.0), axis=0, keepdims=True)
    mw = jnp.sum(jnp.where(onehot, gw, 0.0), axis=0, keepdims=True)
    mh = jnp.sum(jnp.where(onehot, gh, 0.0), axis=0, keepdims=True)
    tgt_cls = jnp.where(pos,
                        jnp.sum(jnp.where(onehot, lblf, 0.0), axis=0,
                                keepdims=True),
                        0.0)

    e0 = (mcx - pcx) / pw
    e1 = (mcy - pcy) / ph
    e2 = jnp.log(mw / pw)
    e3 = jnp.log(mh / ph)
    sl1_acc = jnp.zeros((1, ss, ll), f32)
    for c, enc in enumerate((e0, e1, e2, e3)):
        d = loc[c:c + 1] - enc
        ad = jnp.abs(d)
        sl1 = jnp.where(ad < 1.0, 0.5 * d * d, ad - 0.5)
        sl1_acc = sl1_acc + jnp.where(pos, sl1, 0.0)

    xm = jnp.max(conf, axis=0, keepdims=True)
    lse = jnp.log(jnp.sum(jnp.exp(conf - xm), axis=0, keepdims=True)) + xm
    c_iota = jax.lax.broadcasted_iota(jnp.int32, (num_c, 1, 1), 0).astype(f32)
    tgt_logit = jnp.sum(jnp.where(c_iota == tgt_cls, conf, 0.0), axis=0,
                        keepdims=True)
    ce = lse - tgt_logit

    ce_pos = jnp.where(pos, ce, 0.0)
    ce_neg = jnp.where(jnp.logical_or(pos, jnp.logical_not(valid)), 0.0, ce)
    return sl1_acc, pos.astype(f32), ce_pos, ce_neg


def _prog_body(num_p_real, loc_ref, conf_ref, pri_ref, box_ref, lbl_ref,
               out_ref):
    f32 = jnp.float32
    pri = pri_ref[...]
    num_j = loc_ref.shape[0]
    parts = [
        _one_image(num_p_real, loc_ref[j], conf_ref[j], pri, box_ref[j],
                   lbl_ref[j][..., None])
        for j in range(num_j)
    ]
    sl1 = jnp.concatenate([q[0] for q in parts], axis=0)
    posf = jnp.concatenate([q[1] for q in parts], axis=0)
    ce_pos = jnp.concatenate([q[2] for q in parts], axis=0)
    ce_neg = jnp.concatenate([q[3] for q in parts], axis=0)

    loc_sum = jnp.sum(sl1, axis=(1, 2), keepdims=True)
    pos_cnt = jnp.sum(posf, axis=(1, 2), keepdims=True)
    conf_pos_sum = jnp.sum(ce_pos, axis=(1, 2), keepdims=True)

    k = jnp.maximum(pos_cnt * _NEG_POS_RATIO, 1.0)

    def _bis(_, carry):
        lo, hi = carry
        t = 0.5 * (lo + hi)
        n = jnp.sum((ce_neg > t).astype(f32), axis=(1, 2), keepdims=True)
        gt = n > k
        return jnp.where(gt, t, lo), jnp.where(gt, hi, t)

    lo0 = jnp.zeros_like(k)
    hi0 = jnp.max(ce_neg, axis=(1, 2), keepdims=True)
    _, thr = jax.lax.fori_loop(0, _BISECT_ITERS, _bis, (lo0, hi0))
    above = ce_neg > thr
    n_above = jnp.sum(above.astype(f32), axis=(1, 2), keepdims=True)
    s_above = jnp.sum(jnp.where(above, ce_neg, 0.0), axis=(1, 2),
                      keepdims=True)
    neg_sum = s_above + (k - n_above) * thr

    o_iota = jax.lax.broadcasted_iota(jnp.int32, (1, 1, 8), 2)
    out_ref[...] = jnp.where(
        o_iota == 0, loc_sum,
        jnp.where(o_iota == 1, pos_cnt,
                  jnp.where(o_iota == 2, conf_pos_sum,
                            jnp.where(o_iota == 3, neg_sum, 0.0))))


def kernel(loc_preds, conf_preds, priors, gt_boxes, gt_labels):
    b, p, _ = loc_preds.shape
    c = conf_preds.shape[2]
    g = gt_boxes.shape[1]
    pp = _SUBLANES * _LANES
    pad = pp - p
    ipp = _IMGS_PER_PROG

    loc_t = jnp.pad(jnp.transpose(loc_preds, (0, 2, 1)),
                    ((0, 0), (0, 0), (0, pad))).reshape(b, 4, _SUBLANES,
                                                        _LANES)
    conf_t = jnp.pad(jnp.transpose(conf_preds, (0, 2, 1)),
                     ((0, 0), (0, 0), (0, pad))).reshape(b, c, _SUBLANES,
                                                         _LANES)
    pri_t = jnp.pad(jnp.transpose(priors, (1, 0)),
                    ((0, 0), (0, pad))).reshape(4, _SUBLANES, _LANES)
    lbl = gt_labels.astype(jnp.float32).reshape(b, g, 1)

    sums = pl.pallas_call(
        functools.partial(_prog_body, p),
        grid=(b // ipp,),
        in_specs=[
            pl.BlockSpec((ipp, 4, _SUBLANES, _LANES),
                         lambda i: (i, 0, 0, 0)),
            pl.BlockSpec((ipp, c, _SUBLANES, _LANES),
                         lambda i: (i, 0, 0, 0)),
            pl.BlockSpec((4, _SUBLANES, _LANES), lambda i: (0, 0, 0)),
            pl.BlockSpec((ipp, g, 4), lambda i: (i, 0, 0)),
            pl.BlockSpec((ipp, g, 1), lambda i: (i, 0, 0)),
        ],
        out_specs=pl.BlockSpec((ipp, 1, 8), lambda i: (i, 0, 0)),
        out_shape=jax.ShapeDtypeStruct((b, 1, 8), jnp.float32),
        compiler_params=pltpu.CompilerParams(
            dimension_semantics=("parallel",)),
    )(loc_t, conf_t, pri_t, gt_boxes, lbl)

    loc_sum = jnp.sum(sums[:, 0, 0])
    num_pos = jnp.maximum(jnp.sum(sums[:, 0, 1]), 1.0)
    conf_sum = jnp.sum(sums[:, 0, 2]) + jnp.sum(sums[:, 0, 3])
    return (loc_sum + conf_sum) / num_pos


# speedup vs baseline: 37.8437x; 1.0420x over previous
import functools

import jax
import jax.numpy as jnp
from jax.experimental import pallas as pl
from jax.experimental.pallas import tpu as pltpu

_IOU_THRESHOLD = 0.5
_NEG_POS_RATIO = 3.0
_BISECT_ITERS = 16
_LANES = 128
_SUBLANES = 72
_IMGS_PER_PROG = 8


def _one_image(num_p_real, loc, conf, pri, boxes, lblf):
    f32 = jnp.float32
    _, ss, ll = pri.shape
    num_g = boxes.shape[0]
    num_c = conf.shape[0]

    pcx = pri[0:1]
    pcy = pri[1:2]
    pw = pri[2:3]
    ph = pri[3:4]
    px1 = pcx - pw * 0.5
    py1 = pcy - ph * 0.5
    px2 = pcx + pw * 0.5
    py2 = pcy + ph * 0.5

    gx1 = boxes[:, 0:1][..., None]
    gy1 = boxes[:, 1:2][..., None]
    gx2 = boxes[:, 2:3][..., None]
    gy2 = boxes[:, 3:4][..., None]

    ltx = jnp.maximum(px1, gx1)
    lty = jnp.maximum(py1, gy1)
    rbx = jnp.minimum(px2, gx2)
    rby = jnp.minimum(py2, gy2)
    iw = jnp.maximum(rbx - ltx, 0.0)
    ih = jnp.maximum(rby - lty, 0.0)
    inter = iw * ih
    area_p = (px2 - px1) * (py2 - py1)
    area_g = (gx2 - gx1) * (gy2 - gy1)
    ov = inter / (area_p + area_g - inter)

    s_io = jax.lax.broadcasted_iota(jnp.int32, (1, ss, ll), 1)
    l_io = jax.lax.broadcasted_iota(jnp.int32, (1, ss, ll), 2)
    p_iota = (s_io * ll + l_io).astype(f32)
    valid = p_iota < float(num_p_real)
    g_iota = jax.lax.broadcasted_iota(jnp.int32, (num_g, 1, 1), 0).astype(f32)

    bov = jnp.max(ov, axis=0, keepdims=True)
    bgi = jnp.min(jnp.where(ov == bov, g_iota, float(num_g)), axis=0,
                  keepdims=True)
    m_g = jnp.max(ov, axis=(1, 2), keepdims=True)
    bpi = jnp.min(jnp.where(ov == m_g, p_iota, float(ss * ll)), axis=(1, 2),
                  keepdims=True)

    claimed_g = jnp.max(jnp.where(p_iota == bpi, g_iota, -1.0), axis=0,
                        keepdims=True)
    claimed = claimed_g >= 0.0
    bgi = jnp.where(claimed, claimed_g, bgi)
    bov = jnp.where(claimed, 1.0, bov)

    pos = bov > _IOU_THRESHOLD

    onehot = g_iota == bgi
    gcx = (gx1 + gx2) * 0.5
    gcy = (gy1 + gy2) * 0.5
    gw = gx2 - gx1
    gh = gy2 - gy1
    mcx = jnp.sum(jnp.where(onehot, gcx, 0.0), axis=0, keepdims=True)
    mcy = jnp.sum(jnp.where(onehot, gcy, 0.0), axis=0, keepdims=True)
    mw = jnp.sum(jnp.where(onehot, gw, 0.0), axis=0, keepdims=True)
    mh = jnp.sum(jnp.where(onehot, gh, 0.0), axis=0, keepdims=True)
    tgt_cls = jnp.where(pos,
                        jnp.sum(jnp.where(onehot, lblf, 0.0), axis=0,
                                keepdims=True),
                        0.0)

    e0 = (mcx - pcx) / pw
    e1 = (mcy - pcy) / ph
    e2 = jnp.log(mw / pw)
    e3 = jnp.log(mh / ph)
    sl1_acc = jnp.zeros((1, ss, ll), f32)
    for c, enc in enumerate((e0, e1, e2, e3)):
        d = loc[c:c + 1] - enc
        ad = jnp.abs(d)
        sl1 = jnp.where(ad < 1.0, 0.5 * d * d, ad - 0.5)
        sl1_acc = sl1_acc + jnp.where(pos, sl1, 0.0)

    xm = jnp.max(conf, axis=0, keepdims=True)
    lse = jnp.log(jnp.sum(jnp.exp(conf - xm), axis=0, keepdims=True)) + xm
    c_iota = jax.lax.broadcasted_iota(jnp.int32, (num_c, 1, 1), 0).astype(f32)
    tgt_logit = jnp.sum(jnp.where(c_iota == tgt_cls, conf, 0.0), axis=0,
                        keepdims=True)
    ce = lse - tgt_logit

    ce_pos = jnp.where(pos, ce, 0.0)
    ce_neg = jnp.where(jnp.logical_or(pos, jnp.logical_not(valid)), 0.0, ce)
    return sl1_acc, pos.astype(f32), ce_pos, ce_neg


def _prog_body(num_p_real, loc_ref, conf_ref, pri_ref, box_ref, lbl_ref,
               out_ref):
    f32 = jnp.float32
    pri = pri_ref[...]
    num_j = loc_ref.shape[0]
    parts = [
        _one_image(num_p_real, loc_ref[j], conf_ref[j], pri, box_ref[j],
                   lbl_ref[j][..., None])
        for j in range(num_j)
    ]
    sl1 = jnp.concatenate([q[0] for q in parts], axis=0)
    posf = jnp.concatenate([q[1] for q in parts], axis=0)
    ce_pos = jnp.concatenate([q[2] for q in parts], axis=0)
    ce_neg = jnp.concatenate([q[3] for q in parts], axis=0)

    loc_sum = jnp.sum(sl1, axis=(1, 2), keepdims=True)
    pos_cnt = jnp.sum(posf, axis=(1, 2), keepdims=True)
    conf_pos_sum = jnp.sum(ce_pos, axis=(1, 2), keepdims=True)

    k = jnp.maximum(pos_cnt * _NEG_POS_RATIO, 1.0)

    def _bis(_, carry):
        lo, hi = carry
        t = 0.5 * (lo + hi)
        n = jnp.sum((ce_neg > t).astype(f32), axis=(1, 2), keepdims=True)
        gt = n > k
        return jnp.where(gt, t, lo), jnp.where(gt, hi, t)

    lo0 = jnp.zeros_like(k)
    hi0 = jnp.max(ce_neg, axis=(1, 2), keepdims=True)
    _, thr = jax.lax.fori_loop(0, _BISECT_ITERS, _bis, (lo0, hi0))
    above = ce_neg > thr
    n_above = jnp.sum(above.astype(f32), axis=(1, 2), keepdims=True)
    s_above = jnp.sum(jnp.where(above, ce_neg, 0.0), axis=(1, 2),
                      keepdims=True)
    neg_sum = s_above + (k - n_above) * thr

    o_iota = jax.lax.broadcasted_iota(jnp.int32, (1, 1, 8), 2)
    out_ref[...] = jnp.where(
        o_iota == 0, loc_sum,
        jnp.where(o_iota == 1, pos_cnt,
                  jnp.where(o_iota == 2, conf_pos_sum,
                            jnp.where(o_iota == 3, neg_sum, 0.0))))


def kernel(loc_preds, conf_preds, priors, gt_boxes, gt_labels):
    b, p, _ = loc_preds.shape
    c = conf_preds.shape[2]
    g = gt_boxes.shape[1]
    pp = _SUBLANES * _LANES
    pad = pp - p
    ipp = _IMGS_PER_PROG

    loc_t = jnp.pad(jnp.transpose(loc_preds, (0, 2, 1)),
                    ((0, 0), (0, 0), (0, pad))).reshape(b, 4, _SUBLANES,
                                                        _LANES)
    conf_t = jnp.pad(jnp.transpose(conf_preds, (0, 2, 1)),
                     ((0, 0), (0, 0), (0, pad))).reshape(b, c, _SUBLANES,
                                                         _LANES)
    pri_t = jnp.pad(jnp.transpose(priors, (1, 0)),
                    ((0, 0), (0, pad))).reshape(4, _SUBLANES, _LANES)
    lbl = gt_labels.astype(jnp.float32).reshape(b, g, 1)

    sums = pl.pallas_call(
        functools.partial(_prog_body, p),
        grid=(b // ipp,),
        in_specs=[
            pl.BlockSpec((ipp, 4, _SUBLANES, _LANES),
                         lambda i: (i, 0, 0, 0)),
            pl.BlockSpec((ipp, c, _SUBLANES, _LANES),
                         lambda i: (i, 0, 0, 0)),
            pl.BlockSpec((4, _SUBLANES, _LANES), lambda i: (0, 0, 0)),
            pl.BlockSpec((ipp, g, 4), lambda i: (i, 0, 0)),
            pl.BlockSpec((ipp, g, 1), lambda i: (i, 0, 0)),
        ],
        out_specs=pl.BlockSpec((ipp, 1, 8), lambda i: (i, 0, 0)),
        out_shape=jax.ShapeDtypeStruct((b, 1, 8), jnp.float32),
        compiler_params=pltpu.CompilerParams(
            dimension_semantics=("parallel",)),
    )(loc_t, conf_t, pri_t, gt_boxes, lbl)

    loc_sum = jnp.sum(sums[:, 0, 0])
    num_pos = jnp.maximum(jnp.sum(sums[:, 0, 1]), 1.0)
    conf_sum = jnp.sum(sums[:, 0, 2]) + jnp.sum(sums[:, 0, 3])
    return (loc_sum + conf_sum) / num_pos
